# Initial kernel scaffold; baseline (speedup 1.0000x reference)
import jax
import jax.numpy as jnp
from jax.experimental import pallas as pl


def kernel(idx, table, targets):
    raise NotImplementedError("write your pallas kernel here")


# speedup vs baseline: 1.2521x; 1.2521x over previous
import functools

import jax
import jax.numpy as jnp
from jax.experimental import pallas as pl
from jax.experimental.pallas import tpu as pltpu

_BLOCK_N = 256
_LANES = 128


def _gather_ce_kernel(idx_sref, tgt_ref, table_ref, logits_ref, nll_ref,
                      tile_ref, *, block_n, n_chunks, stride):
    base = pl.program_id(0) * block_n

    for mi in range(block_n):
        i16 = pl.multiple_of(idx_sref[base + mi], n_chunks)
        slab = table_ref[pl.ds(i16, n_chunks), :]
        tile_ref[mi:mi + n_chunks * stride:stride, :] = slab

    for j in range(n_chunks):
        logits_ref[:, j * _LANES:(j + 1) * _LANES] = (
            tile_ref[pl.ds(j * stride, block_n), :])

    logits = logits_ref[...]
    vp = logits.shape[1]
    m = jnp.max(logits, axis=-1, keepdims=True)
    lse = m + jnp.log(jnp.sum(jnp.exp(logits - m), axis=-1, keepdims=True))
    col = jax.lax.broadcasted_iota(jnp.int32, (block_n, vp), 1)
    tgt = tgt_ref[...]
    tgt_logit = jnp.sum(jnp.where(col == tgt, logits, 0.0),
                        axis=-1, keepdims=True)
    nll_ref[...] = lse - tgt_logit


def _bigram_forward(idx, table, targets, *, block_n=_BLOCK_N):
    B, T = idx.shape
    V = table.shape[0]
    N = B * T
    n_chunks = V // _LANES
    num_blocks = N // block_n
    stride = block_n + 8
    tile_rows = (n_chunks - 1) * stride + block_n

    table2 = table.reshape(V * n_chunks, _LANES)
    idx_scaled = (idx.astype(jnp.int32) * n_chunks).reshape(N)
    tgt_col = targets.astype(jnp.int32).reshape(N, 1)

    kern = functools.partial(_gather_ce_kernel, block_n=block_n,
                             n_chunks=n_chunks, stride=stride)

    logits_flat, nll = pl.pallas_call(
        kern,
        grid_spec=pltpu.PrefetchScalarGridSpec(
            num_scalar_prefetch=1,
            grid=(num_blocks,),
            in_specs=[
                pl.BlockSpec((block_n, 1), lambda i, s: (i, 0)),
                pl.BlockSpec((V * n_chunks, _LANES), lambda i, s: (0, 0)),
            ],
            out_specs=(
                pl.BlockSpec((block_n, V), lambda i, s: (i, 0)),
                pl.BlockSpec((block_n, 1), lambda i, s: (i, 0)),
            ),
            scratch_shapes=[pltpu.VMEM((tile_rows, _LANES), jnp.float32)],
        ),
        out_shape=(
            jax.ShapeDtypeStruct((N, V), jnp.float32),
            jax.ShapeDtypeStruct((N, 1), jnp.float32),
        ),
        compiler_params=pltpu.CompilerParams(
            dimension_semantics=("parallel",),
            vmem_limit_bytes=48 * 1024 * 1024,
        ),
        cost_estimate=pl.CostEstimate(
            flops=4 * N * V,
            transcendentals=N * V,
            bytes_accessed=N * V * 4 * 2 + V * V * 4,
        ),
    )(idx_scaled, tgt_col, table2)

    logits = logits_flat.reshape(B, T, V)
    loss = jnp.sum(nll[:, 0]) / N
    return logits, loss


def kernel(idx, table, targets):
    return _bigram_forward(idx, table, targets)
